# baseline (speedup 1.0000x reference)
import functools

import jax
import jax.numpy as jnp
from jax import lax
from jax.experimental import pallas as pl
from jax.experimental.pallas import tpu as pltpu
from jax.experimental.pallas import tpu_sc as plsc

_N = 10000
_E = 320000
_D = 128
_NC = 2
_NS = 16
_NW = _NC * _NS
_NP = 10240
_EROWS = 2560
_RPT = _EROWS // _NW
_K = 2
_NCHUNK = _RPT // _K
_T = _NCHUNK // 2
_EROWS_ST = _EROWS + 2 * _K
_NPS = _NP // _NS
_BF = jnp.bfloat16


def _make_sc_segsum(with_counts: bool):
    mesh = plsc.VectorSubcoreMesh(core_axis_name="c", subcore_axis_name="s")
    out_type = [jax.ShapeDtypeStruct((_NC, _NP, _D), _BF)]
    scratch = [
        pltpu.VMEM((_K * 128,), jnp.int32),
        pltpu.VMEM((_K * 128,), jnp.int32),
        pltpu.VMEM((_K * 128,), jnp.int32),
        pltpu.VMEM((_K * 128,), jnp.int32),
        pltpu.VMEM((_K * 128, _D), _BF),
        pltpu.VMEM((_K * 128, _D), _BF),
        pltpu.VMEM_SHARED((_NP, _D), _BF),
        pltpu.VMEM_SHARED((_NP, _D), _BF),
        pltpu.SemaphoreType.DMA,
        pltpu.SemaphoreType.DMA,
        pltpu.SemaphoreType.DMA,
        pltpu.SemaphoreType.DMA,
        pltpu.SemaphoreType.DMA,
        pltpu.SemaphoreType.DMA,
        pltpu.SemaphoreType.DMA,
        pltpu.SemaphoreType.DMA,
    ]
    if with_counts:
        out_type.append(jax.ShapeDtypeStruct((_NC, _NP, 16), jnp.float32))
        scratch += [
            pltpu.VMEM((_K * 128, 16), jnp.float32),
            pltpu.VMEM_SHARED((_NP, 16), jnp.float32),
        ]

    def body(table, edges, zrow, *rest):
        if with_counts:
            (zcnt, ones_h, sum_out, cnt_out,
             sidx_a, sidx_b, didx_a, didx_b, rows_a, rows_b,
             acc_sh, tab_sh, g_a, g_b, s_a, s_b, is_a, is_b, id_a, id_b,
             ones_v, cnt_sh) = rest
        else:
            (sum_out, sidx_a, sidx_b, didx_a, didx_b, rows_a, rows_b,
             acc_sh, tab_sh, g_a, g_b, s_a, s_b,
             is_a, is_b, id_a, id_b) = rest
        cid = lax.axis_index("c")
        sid = lax.axis_index("s")
        wid = cid * _NS + sid
        sidx = (sidx_a, sidx_b)
        didx = (didx_a, didx_b)
        rows = (rows_a, rows_b)
        gsem = (g_a, g_b)
        ssem = (s_a, s_b)
        isem = (is_a, is_b)
        dsem = (id_a, id_b)

        pltpu.sync_copy(table.at[pl.ds(sid * _NPS, _NPS)],
                        tab_sh.at[pl.ds(sid * _NPS, _NPS)])
        pltpu.sync_copy(zrow, acc_sh.at[pl.ds(sid * _NPS, _NPS)])
        if with_counts:
            pltpu.sync_copy(zcnt, cnt_sh.at[pl.ds(sid * _NPS, _NPS)])
            pltpu.sync_copy(ones_h, ones_v)
        plsc.subcore_barrier()

        row_base = wid * _RPT

        def idx_rows(plane, chunk):
            return edges.at[plane,
                            pl.ds((row_base + chunk * _K) * 128, _K * 128)]

        def sidx_copy(buf, chunk, start):
            cp = pltpu.make_async_copy(idx_rows(0, chunk), sidx[buf],
                                       isem[buf])
            if start:
                cp.start()
            else:
                cp.wait()

        def didx_copy(buf, chunk, start):
            cp = pltpu.make_async_copy(idx_rows(1, chunk), didx[buf],
                                       dsem[buf])
            if start:
                cp.start()
            else:
                cp.wait()

        def gathers(buf, start):
            cp = pltpu.make_async_copy(
                tab_sh.at[sidx[buf]], rows[buf], gsem[buf])
            if start:
                cp.start()
            else:
                cp.wait()

        def scatters(buf, start):
            cp = pltpu.make_async_copy(
                rows[buf], acc_sh.at[didx[buf]], ssem[buf])
            if start:
                cp.start(add=True)
            else:
                cp.wait()
            if with_counts:
                cp2 = pltpu.make_async_copy(ones_v, cnt_sh.at[didx[buf]],
                                            ssem[buf])
                if start:
                    cp2.start(add=True)
                else:
                    cp2.wait()

        pltpu.sync_copy(idx_rows(0, 0), sidx_a)
        gathers(0, start=True)
        didx_copy(0, 0, start=True)
        sidx_copy(1, 1, start=True)
        pltpu.sync_copy(edges.at[1, pl.ds(_EROWS * 128, _K * 128)], didx_b)
        scatters(1, start=True)

        def pipelined(t, carry):
            i0 = 2 * t
            i1 = i0 + 1
            scatters(1, start=False)
            didx_copy(1, i1, start=True)
            sidx_copy(1, i1, start=False)
            gathers(1, start=True)
            gathers(0, start=False)
            sidx_copy(0, i0 + 2, start=True)
            didx_copy(0, i0, start=False)
            scatters(0, start=True)
            scatters(0, start=False)
            didx_copy(0, i0 + 2, start=True)
            sidx_copy(0, i0 + 2, start=False)
            gathers(0, start=True)
            gathers(1, start=False)
            sidx_copy(1, i1 + 2, start=True)
            didx_copy(1, i1, start=False)
            scatters(1, start=True)
            return carry

        lax.fori_loop(0, _T, pipelined, 0)
        gathers(0, start=False)
        scatters(1, start=False)
        sidx_copy(1, 0, start=False)
        didx_copy(0, 0, start=False)
        plsc.subcore_barrier()

        pltpu.sync_copy(acc_sh.at[pl.ds(sid * _NPS, _NPS)],
                        sum_out.at[cid, pl.ds(sid * _NPS, _NPS)])
        if with_counts:
            pltpu.sync_copy(cnt_sh.at[pl.ds(sid * _NPS, _NPS)],
                            cnt_out.at[cid, pl.ds(sid * _NPS, _NPS)])

    return pl.kernel(
        body, mesh=mesh, out_type=out_type, scratch_types=scratch,
        compiler_params=pltpu.CompilerParams(use_tc_tiling_on_sc=False))


_sc_segsum_counts = _make_sc_segsum(True)
_sc_segsum_plain = _make_sc_segsum(False)


def _dense_body(relu, out_bf, p_ref, cnt_ref, x_ref, wl_ref, wr_ref, b_ref,
                o_ref):
    s = p_ref[0].astype(jnp.float32) + p_ref[1].astype(jnp.float32)
    c = cnt_ref[0, :, 0:1] + cnt_ref[1, :, 0:1]
    aggr = s / jnp.clip(c, 1.0, None)
    xf = x_ref[...].astype(jnp.float32)
    out = (jnp.dot(aggr, wl_ref[...], preferred_element_type=jnp.float32)
           + jnp.dot(xf, wr_ref[...], preferred_element_type=jnp.float32)
           + b_ref[...])
    if relu:
        out = jnp.maximum(out, 0.0)
    if out_bf:
        o_ref[...] = out.astype(_BF)
    else:
        o_ref[...] = out


def _dense(p3, cnt3, xbf, wl, wr, b, relu, out_bf, n_out=_NP):
    br = 1024
    grid = _NP // br
    odt = _BF if out_bf else jnp.float32
    return pl.pallas_call(
        functools.partial(_dense_body, relu, out_bf),
        grid=(grid,),
        in_specs=[
            pl.BlockSpec((_NC, br, _D), lambda i: (0, i, 0)),
            pl.BlockSpec((_NC, br, 16), lambda i: (0, i, 0)),
            pl.BlockSpec((br, _D), lambda i: (i, 0)),
            pl.BlockSpec((_D, _D), lambda i: (0, 0)),
            pl.BlockSpec((_D, _D), lambda i: (0, 0)),
            pl.BlockSpec((1, _D), lambda i: (0, 0)),
        ],
        out_specs=pl.BlockSpec((br, _D), lambda i: (i, 0)),
        out_shape=jax.ShapeDtypeStruct((n_out, _D), odt),
    )(p3, cnt3, xbf, wl, wr, b.reshape(1, _D))


def kernel(x, edge_index, W_l1, W_r1, b1, W_l2, W_r2, b2):
    er = edge_index.astype(jnp.int32)
    pad_cols = jnp.broadcast_to(
        jnp.array([0, _NP - 1], jnp.int32)[:, None],
        (2, _EROWS_ST * 128 - _E))
    edges = jnp.concatenate([er, pad_cols], axis=1)
    zrow = jnp.zeros((_NPS, _D), _BF)
    zcnt = jnp.zeros((_NPS, 16), jnp.float32)
    ones = jnp.ones((_K * 128, 16), jnp.float32)

    xbf = jnp.pad(x.astype(_BF), ((0, _NP - _N), (0, 0)))
    s1, cnts = _sc_segsum_counts(xbf, edges, zrow, zcnt, ones)
    hbf = _dense(s1, cnts, xbf, W_l1, W_r1, b1, relu=True, out_bf=True)
    (s2,) = _sc_segsum_plain(hbf, edges, zrow)
    return _dense(s2, cnts, hbf, W_l2, W_r2, b2,
                  relu=False, out_bf=False, n_out=_N)


# speedup vs baseline: 1.0513x; 1.0513x over previous
import functools

import jax
import jax.numpy as jnp
from jax import lax
from jax.experimental import pallas as pl
from jax.experimental.pallas import tpu as pltpu
from jax.experimental.pallas import tpu_sc as plsc

_N = 10000
_E = 320000
_D = 128
_NC = 2
_NS = 16
_NW = _NC * _NS
_NP = 10240
_EROWS = 2560
_RPT = _EROWS // _NW
_K = 2
_NCHUNK = _RPT // _K
_T = _NCHUNK // 2
_EROWS_ST = _EROWS + 2 * _K
_NPS = _NP // _NS
_BF = jnp.bfloat16


def _make_sc_segsum(with_counts: bool):
    mesh = plsc.VectorSubcoreMesh(core_axis_name="c", subcore_axis_name="s")
    out_type = [jax.ShapeDtypeStruct((_NC, _NP, _D), _BF)]
    scratch = [
        pltpu.VMEM((_K, 128), jnp.int32),
        pltpu.VMEM((_K, 128), jnp.int32),
        pltpu.VMEM((_K, 128), jnp.int32),
        pltpu.VMEM((_K, 128), jnp.int32),
        pltpu.VMEM((_K * 128, _D), _BF),
        pltpu.VMEM((_K * 128, _D), _BF),
        pltpu.VMEM_SHARED((_NP, _D), _BF),
        pltpu.VMEM_SHARED((_NP, _D), _BF),
        pltpu.SemaphoreType.DMA,
        pltpu.SemaphoreType.DMA,
        pltpu.SemaphoreType.DMA,
        pltpu.SemaphoreType.DMA,
        pltpu.SemaphoreType.DMA,
        pltpu.SemaphoreType.DMA,
        pltpu.SemaphoreType.DMA,
        pltpu.SemaphoreType.DMA,
    ]
    if with_counts:
        out_type.append(jax.ShapeDtypeStruct((_NC, _NP, 16), jnp.float32))
        scratch += [
            pltpu.VMEM((128, 16), jnp.float32),
            pltpu.VMEM((128, 16), jnp.float32),
            pltpu.VMEM_SHARED((_NP, 16), jnp.float32),
        ]

    def body(table, edges, zrow, *rest):
        if with_counts:
            (zcnt, ones_h, sum_out, cnt_out,
             sidx_a, sidx_b, didx_a, didx_b, rows_a, rows_b,
             acc_sh, tab_sh, g_a, g_b, s_a, s_b, is_a, is_b, id_a, id_b,
             ones_v, zones_v, cnt_sh) = rest
        else:
            (sum_out, sidx_a, sidx_b, didx_a, didx_b, rows_a, rows_b,
             acc_sh, tab_sh, g_a, g_b, s_a, s_b,
             is_a, is_b, id_a, id_b) = rest
        cid = lax.axis_index("c")
        sid = lax.axis_index("s")
        wid = cid * _NS + sid
        sidx = (sidx_a, sidx_b)
        didx = (didx_a, didx_b)
        rows = (rows_a, rows_b)
        gsem = (g_a, g_b)
        ssem = (s_a, s_b)
        isem = (is_a, is_b)
        dsem = (id_a, id_b)

        pltpu.sync_copy(table.at[pl.ds(sid * _NPS, _NPS)],
                        tab_sh.at[pl.ds(sid * _NPS, _NPS)])
        pltpu.sync_copy(zrow, acc_sh.at[pl.ds(sid * _NPS, _NPS)])
        pltpu.sync_copy(zrow.at[pl.ds(0, _K * 128)], rows_b)
        if with_counts:
            pltpu.sync_copy(zcnt, cnt_sh.at[pl.ds(sid * _NPS, _NPS)])
            pltpu.sync_copy(ones_h, ones_v)
            pltpu.sync_copy(zcnt.at[pl.ds(0, 128)], zones_v)
        plsc.subcore_barrier()

        row_base = wid * _RPT

        def idx_rows(plane, chunk):
            return edges.at[plane, pl.ds(row_base + chunk * _K, _K)]

        def sidx_copy(buf, chunk, start):
            cp = pltpu.make_async_copy(idx_rows(0, chunk), sidx[buf],
                                       isem[buf])
            if start:
                cp.start()
            else:
                cp.wait()

        def didx_copy(buf, chunk, start):
            cp = pltpu.make_async_copy(idx_rows(1, chunk), didx[buf],
                                       dsem[buf])
            if start:
                cp.start()
            else:
                cp.wait()

        def gathers(buf, start):
            for j in range(_K):
                cp = pltpu.make_async_copy(
                    tab_sh.at[sidx[buf].at[j]],
                    rows[buf].at[pl.ds(j * 128, 128)], gsem[buf])
                if start:
                    cp.start()
                else:
                    cp.wait()

        def scatters(buf, start, prime=False):
            for j in range(_K):
                dsti = didx[buf].at[j]
                cp = pltpu.make_async_copy(
                    rows[buf].at[pl.ds(j * 128, 128)],
                    acc_sh.at[dsti], ssem[buf])
                if start:
                    cp.start(add=True)
                else:
                    cp.wait()
                if with_counts:
                    csrc = zones_v if prime else ones_v
                    cp2 = pltpu.make_async_copy(csrc, cnt_sh.at[dsti],
                                                ssem[buf])
                    if start:
                        cp2.start(add=True)
                    else:
                        cp2.wait()

        pltpu.sync_copy(idx_rows(0, 0), sidx_a)
        gathers(0, start=True)
        didx_copy(0, 0, start=True)
        sidx_copy(1, 1, start=True)
        pltpu.sync_copy(idx_rows(1, 0), didx_b)
        scatters(1, start=True, prime=True)

        def pipelined(t, carry):
            i0 = 2 * t
            i1 = i0 + 1
            scatters(1, start=False)
            didx_copy(1, i1, start=True)
            sidx_copy(1, i1, start=False)
            gathers(1, start=True)
            gathers(0, start=False)
            sidx_copy(0, i0 + 2, start=True)
            didx_copy(0, i0, start=False)
            scatters(0, start=True)
            scatters(0, start=False)
            didx_copy(0, i0 + 2, start=True)
            sidx_copy(0, i0 + 2, start=False)
            gathers(0, start=True)
            gathers(1, start=False)
            sidx_copy(1, i1 + 2, start=True)
            didx_copy(1, i1, start=False)
            scatters(1, start=True)
            return carry

        lax.fori_loop(0, _T, pipelined, 0)
        gathers(0, start=False)
        scatters(1, start=False)
        sidx_copy(1, 0, start=False)
        didx_copy(0, 0, start=False)
        plsc.subcore_barrier()

        pltpu.sync_copy(acc_sh.at[pl.ds(sid * _NPS, _NPS)],
                        sum_out.at[cid, pl.ds(sid * _NPS, _NPS)])
        if with_counts:
            pltpu.sync_copy(cnt_sh.at[pl.ds(sid * _NPS, _NPS)],
                            cnt_out.at[cid, pl.ds(sid * _NPS, _NPS)])

    return pl.kernel(
        body, mesh=mesh, out_type=out_type, scratch_types=scratch,
        compiler_params=pltpu.CompilerParams(use_tc_tiling_on_sc=False))


_sc_segsum_counts = _make_sc_segsum(True)
_sc_segsum_plain = _make_sc_segsum(False)


def _dense_body(relu, out_bf, p_ref, cnt_ref, x_ref, wl_ref, wr_ref, b_ref,
                o_ref):
    s = p_ref[0].astype(jnp.float32) + p_ref[1].astype(jnp.float32)
    c = cnt_ref[0, :, 0:1] + cnt_ref[1, :, 0:1]
    aggr = s / jnp.clip(c, 1.0, None)
    xf = x_ref[...].astype(jnp.float32)
    out = (jnp.dot(aggr, wl_ref[...], preferred_element_type=jnp.float32)
           + jnp.dot(xf, wr_ref[...], preferred_element_type=jnp.float32)
           + b_ref[...])
    if relu:
        out = jnp.maximum(out, 0.0)
    if out_bf:
        o_ref[...] = out.astype(_BF)
    else:
        o_ref[...] = out


def _dense(p3, cnt3, xbf, wl, wr, b, relu, out_bf, n_out=_NP):
    br = 1024
    grid = _NP // br
    odt = _BF if out_bf else jnp.float32
    return pl.pallas_call(
        functools.partial(_dense_body, relu, out_bf),
        grid=(grid,),
        in_specs=[
            pl.BlockSpec((_NC, br, _D), lambda i: (0, i, 0)),
            pl.BlockSpec((_NC, br, 16), lambda i: (0, i, 0)),
            pl.BlockSpec((br, _D), lambda i: (i, 0)),
            pl.BlockSpec((_D, _D), lambda i: (0, 0)),
            pl.BlockSpec((_D, _D), lambda i: (0, 0)),
            pl.BlockSpec((1, _D), lambda i: (0, 0)),
        ],
        out_specs=pl.BlockSpec((br, _D), lambda i: (i, 0)),
        out_shape=jax.ShapeDtypeStruct((n_out, _D), odt),
    )(p3, cnt3, xbf, wl, wr, b.reshape(1, _D))


def kernel(x, edge_index, W_l1, W_r1, b1, W_l2, W_r2, b2):
    er = edge_index.astype(jnp.int32).reshape(2, _E // 128, 128)
    pad_rows = jnp.broadcast_to(
        jnp.array([0, _NP - 1], jnp.int32)[:, None, None],
        (2, _EROWS_ST - _E // 128, 128))
    edges = jnp.concatenate([er, pad_rows], axis=1)
    zrow = jnp.zeros((_NPS, _D), _BF)
    zcnt = jnp.zeros((_NPS, 16), jnp.float32)
    ones = jnp.ones((128, 16), jnp.float32)

    xbf = jnp.pad(x.astype(_BF), ((0, _NP - _N), (0, 0)))
    s1, cnts = _sc_segsum_counts(xbf, edges, zrow, zcnt, ones)
    hbf = _dense(s1, cnts, xbf, W_l1, W_r1, b1, relu=True, out_bf=True)
    (s2,) = _sc_segsum_plain(hbf, edges, zrow)
    return _dense(s2, cnts, hbf, W_l2, W_r2, b2,
                  relu=False, out_bf=False, n_out=_N)


# speedup vs baseline: 1.0637x; 1.0117x over previous
import functools

import jax
import jax.numpy as jnp
from jax import lax
from jax.experimental import pallas as pl
from jax.experimental.pallas import tpu as pltpu
from jax.experimental.pallas import tpu_sc as plsc

_N = 10000
_E = 320000
_D = 128
_NC = 2
_NS = 16
_NW = _NC * _NS
_NP = 10240
_EROWS = 2560
_RPT = _EROWS // _NW
_K = 2
_NCHUNK = _RPT // _K
_T = _NCHUNK // 2
_EROWS_ST = _EROWS + 2 * _K
_NPS = _NP // _NS
_BF = jnp.bfloat16


def _make_sc_segsum(with_counts: bool):
    mesh = plsc.VectorSubcoreMesh(core_axis_name="c", subcore_axis_name="s")
    out_type = [jax.ShapeDtypeStruct((_NC, _NP, _D), _BF)]
    scratch = [
        pltpu.VMEM((_K, 128), jnp.int32),
        pltpu.VMEM((_K, 128), jnp.int32),
        pltpu.VMEM((_K, 128), jnp.int32),
        pltpu.VMEM((_K, 128), jnp.int32),
        pltpu.VMEM((_K * 128, _D), _BF),
        pltpu.VMEM((_K * 128, _D), _BF),
        pltpu.VMEM_SHARED((_NP, _D), _BF),
        pltpu.VMEM_SHARED((_NP, _D), _BF),
        pltpu.SemaphoreType.DMA,
        pltpu.SemaphoreType.DMA,
        pltpu.SemaphoreType.DMA,
        pltpu.SemaphoreType.DMA,
        pltpu.SemaphoreType.DMA,
        pltpu.SemaphoreType.DMA,
        pltpu.SemaphoreType.DMA,
        pltpu.SemaphoreType.DMA,
    ]
    if with_counts:
        out_type.append(jax.ShapeDtypeStruct((_NC, _NP, 16), jnp.float32))
        scratch += [
            pltpu.VMEM((128, 16), jnp.float32),
            pltpu.VMEM((128, 16), jnp.float32),
            pltpu.VMEM_SHARED((_NP, 16), jnp.float32),
        ]

    def body(table, edges, zrow, *rest):
        if with_counts:
            (zcnt, ones_h, sum_out, cnt_out,
             sidx_a, sidx_b, didx_a, didx_b, rows_a, rows_b,
             acc_sh, tab_sh, g_a, g_b, s_a, s_b, is_a, is_b, id_a, id_b,
             ones_v, zones_v, cnt_sh) = rest
        else:
            (sum_out, sidx_a, sidx_b, didx_a, didx_b, rows_a, rows_b,
             acc_sh, tab_sh, g_a, g_b, s_a, s_b,
             is_a, is_b, id_a, id_b) = rest
        cid = lax.axis_index("c")
        sid = lax.axis_index("s")
        wid = cid * _NS + sid
        sidx = (sidx_a, sidx_b)
        didx = (didx_a, didx_b)
        rows = (rows_a, rows_b)
        gsem = (g_a, g_b)
        ssem = (s_a, s_b)
        isem = (is_a, is_b)
        dsem = (id_a, id_b)

        pltpu.sync_copy(table.at[pl.ds(sid * _NPS, _NPS)],
                        tab_sh.at[pl.ds(sid * _NPS, _NPS)])
        pltpu.sync_copy(zrow, acc_sh.at[pl.ds(sid * _NPS, _NPS)])
        pltpu.sync_copy(zrow.at[pl.ds(0, _K * 128)], rows_b)
        if with_counts:
            pltpu.sync_copy(zcnt, cnt_sh.at[pl.ds(sid * _NPS, _NPS)])
            pltpu.sync_copy(ones_h, ones_v)
            pltpu.sync_copy(zcnt.at[pl.ds(0, 128)], zones_v)
        plsc.subcore_barrier()

        row_base = wid * _RPT

        def idx_rows(plane, chunk):
            return edges.at[plane, pl.ds(row_base + chunk * _K, _K)]

        def sidx_copy(buf, chunk, start):
            cp = pltpu.make_async_copy(idx_rows(0, chunk), sidx[buf],
                                       isem[buf])
            if start:
                cp.start()
            else:
                cp.wait()

        def didx_copy(buf, chunk, start):
            cp = pltpu.make_async_copy(idx_rows(1, chunk), didx[buf],
                                       dsem[buf])
            if start:
                cp.start()
            else:
                cp.wait()

        def gathers(buf, start):
            for j in range(_K):
                cp = pltpu.make_async_copy(
                    tab_sh.at[sidx[buf].at[j]],
                    rows[buf].at[pl.ds(j * 128, 128)], gsem[buf])
                if start:
                    cp.start()
                else:
                    cp.wait()

        def scatters(buf, start, prime=False):
            for j in range(_K):
                dsti = didx[buf].at[j]
                cp = pltpu.make_async_copy(
                    rows[buf].at[pl.ds(j * 128, 128)],
                    acc_sh.at[dsti], ssem[buf])
                if start:
                    cp.start(add=True)
                else:
                    cp.wait()
                if with_counts:
                    csrc = zones_v if prime else ones_v
                    cp2 = pltpu.make_async_copy(csrc, cnt_sh.at[dsti],
                                                ssem[buf])
                    if start:
                        cp2.start(add=True)
                    else:
                        cp2.wait()

        pltpu.sync_copy(idx_rows(0, 0), sidx_a)
        gathers(0, start=True)
        didx_copy(0, 0, start=True)
        sidx_copy(1, 1, start=True)
        pltpu.sync_copy(idx_rows(1, 0), didx_b)
        scatters(1, start=True, prime=True)

        def pipelined(t, carry):
            i0 = 2 * t
            i1 = i0 + 1
            scatters(1, start=False)
            didx_copy(1, i1, start=True)
            sidx_copy(1, i1, start=False)
            gathers(1, start=True)
            gathers(0, start=False)
            sidx_copy(0, i0 + 2, start=True)
            didx_copy(0, i0, start=False)
            scatters(0, start=True)
            scatters(0, start=False)
            didx_copy(0, i0 + 2, start=True)
            sidx_copy(0, i0 + 2, start=False)
            gathers(0, start=True)
            gathers(1, start=False)
            sidx_copy(1, i1 + 2, start=True)
            didx_copy(1, i1, start=False)
            scatters(1, start=True)
            return carry

        lax.fori_loop(0, _T, pipelined, 0)
        gathers(0, start=False)
        scatters(1, start=False)
        sidx_copy(1, 0, start=False)
        didx_copy(0, 0, start=False)
        plsc.subcore_barrier()

        pltpu.sync_copy(acc_sh.at[pl.ds(sid * _NPS, _NPS)],
                        sum_out.at[cid, pl.ds(sid * _NPS, _NPS)])
        if with_counts:
            pltpu.sync_copy(cnt_sh.at[pl.ds(sid * _NPS, _NPS)],
                            cnt_out.at[cid, pl.ds(sid * _NPS, _NPS)])

    return pl.kernel(
        body, mesh=mesh, out_type=out_type, scratch_types=scratch,
        compiler_params=pltpu.CompilerParams(use_tc_tiling_on_sc=False))


_sc_segsum_counts = _make_sc_segsum(True)
_sc_segsum_plain = _make_sc_segsum(False)


def _dense_body(relu, out_bf, p_ref, cnt_ref, x_ref, wl_ref, wr_ref, b_ref,
                o_ref):
    s = p_ref[0].astype(jnp.float32) + p_ref[1].astype(jnp.float32)
    c = cnt_ref[0, :, 0:1] + cnt_ref[1, :, 0:1]
    aggr = s / jnp.clip(c, 1.0, None)
    xf = x_ref[...].astype(jnp.float32)
    out = (jnp.dot(aggr, wl_ref[...], preferred_element_type=jnp.float32)
           + jnp.dot(xf, wr_ref[...], preferred_element_type=jnp.float32)
           + b_ref[...])
    if relu:
        out = jnp.maximum(out, 0.0)
    if out_bf:
        o_ref[...] = out.astype(_BF)
    else:
        o_ref[...] = out


def _dense(p3, cnt3, xbf, wl, wr, b, relu, out_bf, n_out=_NP):
    br = 2048
    grid = _NP // br
    odt = _BF if out_bf else jnp.float32
    return pl.pallas_call(
        functools.partial(_dense_body, relu, out_bf),
        grid=(grid,),
        in_specs=[
            pl.BlockSpec((_NC, br, _D), lambda i: (0, i, 0)),
            pl.BlockSpec((_NC, br, 16), lambda i: (0, i, 0)),
            pl.BlockSpec((br, _D), lambda i: (i, 0)),
            pl.BlockSpec((_D, _D), lambda i: (0, 0)),
            pl.BlockSpec((_D, _D), lambda i: (0, 0)),
            pl.BlockSpec((1, _D), lambda i: (0, 0)),
        ],
        out_specs=pl.BlockSpec((br, _D), lambda i: (i, 0)),
        out_shape=jax.ShapeDtypeStruct((n_out, _D), odt),
    )(p3, cnt3, xbf, wl, wr, b.reshape(1, _D))


def kernel(x, edge_index, W_l1, W_r1, b1, W_l2, W_r2, b2):
    er = edge_index.astype(jnp.int32).reshape(2, _E // 128, 128)
    pad_rows = jnp.broadcast_to(
        jnp.array([0, _NP - 1], jnp.int32)[:, None, None],
        (2, _EROWS_ST - _E // 128, 128))
    edges = jnp.concatenate([er, pad_rows], axis=1)
    zrow = jnp.zeros((_NPS, _D), _BF)
    zcnt = jnp.zeros((_NPS, 16), jnp.float32)
    ones = jnp.ones((128, 16), jnp.float32)

    xbf = jnp.pad(x.astype(_BF), ((0, _NP - _N), (0, 0)))
    s1, cnts = _sc_segsum_counts(xbf, edges, zrow, zcnt, ones)
    hbf = _dense(s1, cnts, xbf, W_l1, W_r1, b1, relu=True, out_bf=True)
    (s2,) = _sc_segsum_plain(hbf, edges, zrow)
    return _dense(s2, cnts, hbf, W_l2, W_r2, b2,
                  relu=False, out_bf=False, n_out=_N)
